# baseline (speedup 1.0000x reference)
import functools
import jax
import jax.numpy as jnp
from jax import lax
from jax.experimental import pallas as pl
from jax.experimental.pallas import tpu as pltpu
from jax.experimental.pallas import tpu_sc as plsc

F32 = jnp.float32
D = 128
ADP = 128
CATW = D + ADP
NW = 32
def _pad_rows(n, ch):
    q = NW * ch
    return ((n + q - 1) // q) * q


def _t1_body(wdx_ref, wrx_ref, dxa1_ref, dxa2_ref, dxb_ref,
             rxa1_ref, rxa2_ref, rxb_ref,
             p1dx_ref, catdx_ref, p1rx_ref, catrx_ref):
    wdx = wdx_ref[...]
    p1dx_ref[...] = jnp.dot(wdx, dxa1_ref[...], preferred_element_type=F32)
    catdx_ref[:, :D] = wdx
    catdx_ref[:, D:] = jnp.dot(wdx, dxa2_ref[...], preferred_element_type=F32) + dxb_ref[...]
    wrx = wrx_ref[...]
    p1rx_ref[...] = jnp.dot(wrx, rxa1_ref[...], preferred_element_type=F32)
    catrx_ref[:, :D] = wrx
    catrx_ref[:, D:] = jnp.dot(wrx, rxa2_ref[...], preferred_element_type=F32) + rxb_ref[...]


def _t1(wdx, wrx, dxa1, dxa2, dxb, rxa1, rxa2, rxb):
    vdx, vrx = wdx.shape[0], wrx.shape[0]
    return pl.pallas_call(
        _t1_body,
        out_shape=[
            jax.ShapeDtypeStruct((vdx, ADP), F32),
            jax.ShapeDtypeStruct((vdx, CATW), F32),
            jax.ShapeDtypeStruct((vrx, ADP), F32),
            jax.ShapeDtypeStruct((vrx, CATW), F32),
        ],
    )(wdx, wrx, dxa1, dxa2, dxb, rxa1, rxa2, rxb)


def _run_spec(tab, idx3d, out, idxbuf, rows, gsem, wsem, wid, k, ch):
    base = wid * k
    pltpu.sync_copy(idx3d.at[wid], idxbuf)

    def wait(par, sem):
        pltpu.make_async_copy(out.at[0], rows.at[par], sem.at[par]).wait()

    pltpu.async_copy(tab.at[idxbuf.at[0]], rows.at[0], gsem.at[0])

    def body(c, carry):
        par = lax.rem(c, 2)
        prv = lax.rem(c + 1, 2)

        @pl.when(c >= 2)
        def _():
            wait(par, wsem)

        pltpu.async_copy(tab.at[idxbuf.at[c]], rows.at[par], gsem.at[par])
        wait(prv, gsem)
        pltpu.async_copy(rows.at[prv], out.at[base + c - 1], wsem.at[prv])
        return carry

    lax.fori_loop(1, k, body, 0)
    last = (k - 1) % 2
    wait(last, gsem)
    pltpu.async_copy(rows.at[last], out.at[base + k - 1], wsem.at[last])
    wait((k - 2) % 2, wsem)
    wait(last, wsem)


def _sc_gather_body(widths, chunks, ch, *refs):
    n = len(widths)
    tabs = refs[:n]
    idxs = refs[n:2 * n]
    outs = refs[2 * n:3 * n]
    scratch = list(refs[3 * n:])
    wid = lax.axis_index("s") * 2 + lax.axis_index("c")
    idxbufs = scratch[:n]
    per_w = {}
    for w in sorted(set(widths)):
        per_w[w] = tuple(scratch[n:n + 3])
        del scratch[n:n + 3]
    for i in range(n):
        rows, gsem, wsem = per_w[widths[i]]
        _run_spec(tabs[i], idxs[i], outs[i], idxbufs[i], rows, gsem, wsem,
                  wid, chunks[i], ch)


def _sc_gathers(tables, idx_lists, ch):
    widths = tuple(int(t.shape[1]) for t in tables)
    chunks = tuple(int(ix.shape[1]) for ix in idx_lists)
    scratch = [pltpu.VMEM((k, ch), jnp.int32) for k in chunks]
    for w in sorted(set(widths)):
        scratch.append(pltpu.VMEM((2, ch, w), F32))
        scratch.append(pltpu.SemaphoreType.DMA((2,)))
        scratch.append(pltpu.SemaphoreType.DMA((2,)))
    out_type = [jax.ShapeDtypeStruct((NW * k, ch, w), F32)
                for k, w in zip(chunks, widths)]
    mesh = plsc.VectorSubcoreMesh(core_axis_name="c", subcore_axis_name="s")
    k = pl.kernel(
        functools.partial(_sc_gather_body, widths, chunks, ch),
        out_type=out_type,
        mesh=mesh,
        scratch_types=scratch,
    )
    return k(*tables, *idx_lists)


def _t2_body(nvalid, lb, g1_ref, cat_ref, u_ref, out_ref):
    g1 = g1_ref[...]
    cat = cat_ref[...]
    ea = cat[:, :, :D]
    p2 = cat[:, :, D:]
    t = jnp.tanh(g1 + p2)
    pre = jnp.sum(t * u_ref[...], axis=2)
    m = jnp.max(pre, axis=1, keepdims=True)
    e = jnp.exp(pre - m)
    attn = e / jnp.sum(e, axis=1, keepdims=True)
    res = jnp.sum(attn[:, :, None] * ea, axis=1)
    rid = pl.program_id(0) * lb + lax.broadcasted_iota(jnp.int32, (lb, 1), 0)
    out_ref[...] = jnp.where(rid < nvalid, res, 0.0)


def _t2(g1, cat, u, maxa, nvalid, lb=256):
    nl = g1.shape[0] // maxa
    g1 = g1.reshape(nl, maxa, ADP)
    cat = cat.reshape(nl, maxa, CATW)
    grid = nl // lb
    return pl.pallas_call(
        functools.partial(_t2_body, nvalid, lb),
        grid=(grid,),
        in_specs=[
            pl.BlockSpec((lb, maxa, ADP), lambda i: (i, 0, 0)),
            pl.BlockSpec((lb, maxa, CATW), lambda i: (i, 0, 0)),
            pl.BlockSpec((1, 1, ADP), lambda i: (0, 0, 0)),
        ],
        out_specs=pl.BlockSpec((lb, D), lambda i: (i, 0)),
        out_shape=jax.ShapeDtypeStruct((nl, D), F32),
    )(g1, cat, u.reshape(1, 1, ADP))


def _t3_body(dx_ref, rx_ref, out_ref):
    sdx = jnp.sum(dx_ref[...], axis=1)
    srx = jnp.sum(rx_ref[...], axis=1)
    ndx = sdx * lax.rsqrt(jnp.maximum(jnp.sum(sdx * sdx, axis=1, keepdims=True), 1e-24))
    nrx = srx * lax.rsqrt(jnp.maximum(jnp.sum(srx * srx, axis=1, keepdims=True), 1e-24))
    out_ref[...] = ndx + nrx


def _t3(gdx, grx, ndx, nrx, rows, rb=64):
    gdx = gdx.reshape(rows, ndx, D)
    grx = grx.reshape(rows, nrx, D)
    return pl.pallas_call(
        _t3_body,
        grid=(rows // rb,),
        in_specs=[
            pl.BlockSpec((rb, ndx, D), lambda i: (i, 0, 0)),
            pl.BlockSpec((rb, nrx, D), lambda i: (i, 0, 0)),
        ],
        out_specs=pl.BlockSpec((rb, D), lambda i: (i, 0)),
        out_shape=jax.ShapeDtypeStruct((rows, D), F32),
    )(gdx, grx)


def _t4_body(x_ref, w_ref, b_ref, out_ref):
    y = jnp.dot(x_ref[...], w_ref[...], preferred_element_type=F32) + b_ref[...]
    m = jnp.max(y, axis=1, keepdims=True)
    e = jnp.exp(y - m)
    out_ref[...] = e / jnp.sum(e, axis=1, keepdims=True)


def _t4(x, w, b, rb=64):
    rows, nv = x.shape[0], w.shape[1]
    return pl.pallas_call(
        _t4_body,
        grid=(rows // rb,),
        in_specs=[
            pl.BlockSpec((rb, D), lambda i: (i, 0)),
            pl.BlockSpec((D, nv), lambda i: (0, 0)),
            pl.BlockSpec((1, nv), lambda i: (0, 0)),
        ],
        out_specs=pl.BlockSpec((rb, nv), lambda i: (i, 0)),
        out_shape=jax.ShapeDtypeStruct((rows, nv), F32),
    )(x, w, b.reshape(1, nv))


def _t5_body(oh_ref, tab_ref, out_ref):
    out_ref[0] = jnp.dot(oh_ref[0], tab_ref[...], preferred_element_type=F32)


def _t5(onehot, table):
    v, b, nv = onehot.shape
    return pl.pallas_call(
        _t5_body,
        grid=(v,),
        in_specs=[
            pl.BlockSpec((1, b, nv), lambda i: (i, 0, 0)),
            pl.BlockSpec((nv, D), lambda i: (0, 0)),
        ],
        out_specs=pl.BlockSpec((1, b, D), lambda i: (i, 0, 0)),
        out_shape=jax.ShapeDtypeStruct((v, b, D), F32),
    )(onehot, table)


def kernel(dxseqs, drugseqs, dx_onehot, drug_onehot, dxLeavesList,
           dxAncestorsList, drugLeavesList, drugAncestorsList,
           ctd_dx_leaves_list, ctd_dx_ancesster_list, ctd_dx_rel_list,
           ctd_dx_permute_list, ctd_rx_leaves_list, ctd_rx_ancesster_list,
           ctd_rx_rel_list, ctd_rx_permute_list, EHRdxEmb_W, EHRdrugEmb_W,
           dxOntoW, drugOntoW, dxAttnW, dxAttnb, dxAttnU, drugAttnW,
           drugAttnb, drugAttnU, cooccurW, cooccurB):
    B, V, NDX = dxseqs.shape
    NRX = drugseqs.shape[2]
    DXV, MAXA = dxLeavesList.shape
    RXV = drugLeavesList.shape[0]
    AD = dxAttnW.shape[1]

    def padw(m):
        return jnp.pad(m, ((0, 0), (0, ADP - AD)))

    dxa1, dxa2 = padw(dxAttnW[:D]), padw(dxAttnW[D:])
    rxa1, rxa2 = padw(drugAttnW[:D]), padw(drugAttnW[D:])
    dxb = jnp.pad(dxAttnb, (0, ADP - AD)).reshape(1, ADP)
    rxb = jnp.pad(drugAttnb, (0, ADP - AD)).reshape(1, ADP)
    dxu = jnp.pad(dxAttnU[:, 0], (0, ADP - AD))
    rxu = jnp.pad(drugAttnU[:, 0], (0, ADP - AD))

    p1dx, catdx, p1rx, catrx = _t1(dxOntoW, drugOntoW, dxa1, dxa2, dxb,
                                   rxa1, rxa2, rxb)

    def flatpad(ix, n, ch):
        f = ix.reshape(-1).astype(jnp.int32)
        return jnp.pad(f, (0, n - f.shape[0])).reshape(NW, -1, ch)

    npair_dx = _pad_rows(DXV * MAXA, 80)
    npair_rx = _pad_rows(RXV * MAXA, 80)
    g1dx, gcatdx, g1rx, gcatrx = _sc_gathers(
        [p1dx, catdx, p1rx, catrx],
        [flatpad(dxLeavesList, npair_dx, 80), flatpad(dxAncestorsList, npair_dx, 80),
         flatpad(drugLeavesList, npair_rx, 80), flatpad(drugAncestorsList, npair_rx, 80)],
        ch=80,
    )

    dxall = _t2(g1dx.reshape(-1, ADP), gcatdx.reshape(-1, CATW), dxu,
                MAXA, DXV)
    rxall = _t2(g1rx.reshape(-1, ADP), gcatrx.reshape(-1, CATW), rxu,
                MAXA, RXV)

    nseq_dx = _pad_rows(B * V * NDX, 128)
    nseq_rx = _pad_rows(B * V * NRX, 128)
    seqdx_ix = flatpad(dxseqs, nseq_dx, 128)
    seqrx_ix = flatpad(drugseqs, nseq_rx, 128)
    gehr_dx, gall_dx, gehr_rx, gall_rx = _sc_gathers(
        [EHRdxEmb_W, dxall, EHRdrugEmb_W, rxall],
        [seqdx_ix, seqdx_ix, seqrx_ix, seqrx_ix],
        ch=128,
    )

    EHRVEmb = _t3(gehr_dx.reshape(-1, D)[:B * V * NDX],
                  gehr_rx.reshape(-1, D)[:B * V * NRX], NDX, NRX, B * V)

    cooccurU = _t4(EHRVEmb, cooccurW, cooccurB).reshape(B, V, -1)

    dxontoV = _t5(dx_onehot, dxall[:DXV])
    rxontoV = _t5(drug_onehot, rxall[:RXV])

    ontoEmb = jnp.concatenate(
        [gall_dx.reshape(-1, D)[:B * V * NDX].reshape(B, V, NDX, D),
         gall_rx.reshape(-1, D)[:B * V * NRX].reshape(B, V, NRX, D)], axis=2)

    return (cooccurU,
            EHRVEmb.reshape(B, V, D),
            ontoEmb,
            jnp.transpose(dxontoV, (1, 0, 2)),
            jnp.transpose(rxontoV, (1, 0, 2)))


# speedup vs baseline: 1.2948x; 1.2948x over previous
import functools
import jax
import jax.numpy as jnp
from jax import lax
from jax.experimental import pallas as pl
from jax.experimental.pallas import tpu as pltpu
from jax.experimental.pallas import tpu_sc as plsc

F32 = jnp.float32
D = 128
ADP = 128
CATW = D + ADP
NW = 32
def _pad_rows(n, ch):
    q = NW * ch
    return ((n + q - 1) // q) * q


def _t1_body(wdx_ref, wrx_ref, dxa1_ref, dxa2_ref, dxb_ref,
             rxa1_ref, rxa2_ref, rxb_ref,
             p1dx_ref, catdx_ref, p1rx_ref, catrx_ref):
    wdx = wdx_ref[...]
    p1dx_ref[...] = jnp.dot(wdx, dxa1_ref[...], preferred_element_type=F32)
    catdx_ref[:, :D] = wdx
    catdx_ref[:, D:] = jnp.dot(wdx, dxa2_ref[...], preferred_element_type=F32) + dxb_ref[...]
    wrx = wrx_ref[...]
    p1rx_ref[...] = jnp.dot(wrx, rxa1_ref[...], preferred_element_type=F32)
    catrx_ref[:, :D] = wrx
    catrx_ref[:, D:] = jnp.dot(wrx, rxa2_ref[...], preferred_element_type=F32) + rxb_ref[...]


def _t1(wdx, wrx, dxa1, dxa2, dxb, rxa1, rxa2, rxb):
    vdx, vrx = wdx.shape[0], wrx.shape[0]
    return pl.pallas_call(
        _t1_body,
        out_shape=[
            jax.ShapeDtypeStruct((vdx, ADP), F32),
            jax.ShapeDtypeStruct((vdx, CATW), F32),
            jax.ShapeDtypeStruct((vrx, ADP), F32),
            jax.ShapeDtypeStruct((vrx, CATW), F32),
        ],
    )(wdx, wrx, dxa1, dxa2, dxb, rxa1, rxa2, rxb)


def _run_spec(tab, idx3d, out, idxbuf, rows, gsem, wsem, wid, k, ch):
    base = wid * k
    pltpu.sync_copy(idx3d.at[wid], idxbuf)

    def wait(par, sem):
        pltpu.make_async_copy(out.at[0], rows.at[par], sem.at[par]).wait()

    pltpu.async_copy(tab.at[idxbuf.at[0]], rows.at[0], gsem.at[0])

    def body(c, carry):
        par = lax.rem(c, 2)
        prv = lax.rem(c + 1, 2)

        @pl.when(c >= 2)
        def _():
            wait(par, wsem)

        pltpu.async_copy(tab.at[idxbuf.at[c]], rows.at[par], gsem.at[par])
        wait(prv, gsem)
        pltpu.async_copy(rows.at[prv], out.at[base + c - 1], wsem.at[prv])
        return carry

    lax.fori_loop(1, k, body, 0)
    last = (k - 1) % 2
    wait(last, gsem)
    pltpu.async_copy(rows.at[last], out.at[base + k - 1], wsem.at[last])
    wait((k - 2) % 2, wsem)
    wait(last, wsem)


def _sc_gather_body(widths, chunks, ch, *refs):
    n = len(widths)
    tabs = refs[:n]
    idxs = refs[n:2 * n]
    outs = refs[2 * n:3 * n]
    scratch = list(refs[3 * n:])
    wid = lax.axis_index("s") * 2 + lax.axis_index("c")
    idxbufs = scratch[:n]
    per_w = {}
    for w in sorted(set(widths)):
        per_w[w] = tuple(scratch[n:n + 3])
        del scratch[n:n + 3]
    for i in range(n):
        rows, gsem, wsem = per_w[widths[i]]
        _run_spec(tabs[i], idxs[i], outs[i], idxbufs[i], rows, gsem, wsem,
                  wid, chunks[i], ch)


def _sc_gathers(tables, idx_lists, ch):
    widths = tuple(int(t.shape[1]) for t in tables)
    chunks = tuple(int(ix.shape[1]) for ix in idx_lists)
    scratch = [pltpu.VMEM((k, ch), jnp.int32) for k in chunks]
    for w in sorted(set(widths)):
        scratch.append(pltpu.VMEM((2, ch, w), F32))
        scratch.append(pltpu.SemaphoreType.DMA((2,)))
        scratch.append(pltpu.SemaphoreType.DMA((2,)))
    out_type = [jax.ShapeDtypeStruct((NW * k, ch, w), F32)
                for k, w in zip(chunks, widths)]
    mesh = plsc.VectorSubcoreMesh(core_axis_name="c", subcore_axis_name="s")
    k = pl.kernel(
        functools.partial(_sc_gather_body, widths, chunks, ch),
        out_type=out_type,
        mesh=mesh,
        scratch_types=scratch,
    )
    return k(*tables, *idx_lists)


def _t2_body(nvalid, lb, g1_ref, cat_ref, u_ref, out_ref):
    g1 = g1_ref[...]
    cat = cat_ref[...]
    ea = cat[:, :, :D]
    p2 = cat[:, :, D:]
    t = jnp.tanh(g1 + p2)
    pre = jnp.sum(t * u_ref[...], axis=2)
    m = jnp.max(pre, axis=1, keepdims=True)
    e = jnp.exp(pre - m)
    attn = e / jnp.sum(e, axis=1, keepdims=True)
    res = jnp.sum(attn[:, :, None] * ea, axis=1)
    rid = pl.program_id(0) * lb + lax.broadcasted_iota(jnp.int32, (lb, 1), 0)
    out_ref[...] = jnp.where(rid < nvalid, res, 0.0)


def _t2(g1, cat, u, maxa, nvalid, lb=256):
    nl = g1.shape[0] // maxa
    g1 = g1.reshape(nl, maxa, ADP)
    cat = cat.reshape(nl, maxa, CATW)
    grid = nl // lb
    return pl.pallas_call(
        functools.partial(_t2_body, nvalid, lb),
        grid=(grid,),
        in_specs=[
            pl.BlockSpec((lb, maxa, ADP), lambda i: (i, 0, 0)),
            pl.BlockSpec((lb, maxa, CATW), lambda i: (i, 0, 0)),
            pl.BlockSpec((1, 1, ADP), lambda i: (0, 0, 0)),
        ],
        out_specs=pl.BlockSpec((lb, D), lambda i: (i, 0)),
        out_shape=jax.ShapeDtypeStruct((nl, D), F32),
    )(g1, cat, u.reshape(1, 1, ADP))


def _t3_body(dx_ref, rx_ref, out_ref):
    sdx = jnp.sum(dx_ref[...], axis=1)
    srx = jnp.sum(rx_ref[...], axis=1)
    ndx = sdx * lax.rsqrt(jnp.maximum(jnp.sum(sdx * sdx, axis=1, keepdims=True), 1e-24))
    nrx = srx * lax.rsqrt(jnp.maximum(jnp.sum(srx * srx, axis=1, keepdims=True), 1e-24))
    out_ref[...] = ndx + nrx


def _t3(gdx, grx, ndx, nrx, rows, rb=64):
    gdx = gdx.reshape(rows, ndx, D)
    grx = grx.reshape(rows, nrx, D)
    return pl.pallas_call(
        _t3_body,
        grid=(rows // rb,),
        in_specs=[
            pl.BlockSpec((rb, ndx, D), lambda i: (i, 0, 0)),
            pl.BlockSpec((rb, nrx, D), lambda i: (i, 0, 0)),
        ],
        out_specs=pl.BlockSpec((rb, D), lambda i: (i, 0)),
        out_shape=jax.ShapeDtypeStruct((rows, D), F32),
    )(gdx, grx)


def _t4_body(x_ref, w_ref, b_ref, out_ref):
    y = jnp.dot(x_ref[...], w_ref[...], preferred_element_type=F32) + b_ref[...]
    m = jnp.max(y, axis=1, keepdims=True)
    e = jnp.exp(y - m)
    out_ref[...] = e / jnp.sum(e, axis=1, keepdims=True)


def _t4(x, w, b, rb=64):
    rows, nv = x.shape[0], w.shape[1]
    return pl.pallas_call(
        _t4_body,
        grid=(rows // rb,),
        in_specs=[
            pl.BlockSpec((rb, D), lambda i: (i, 0)),
            pl.BlockSpec((D, nv), lambda i: (0, 0)),
            pl.BlockSpec((1, nv), lambda i: (0, 0)),
        ],
        out_specs=pl.BlockSpec((rb, nv), lambda i: (i, 0)),
        out_shape=jax.ShapeDtypeStruct((rows, nv), F32),
    )(x, w, b.reshape(1, nv))


def _t5_body(oh_ref, tab_ref, out_ref):
    out_ref[0] = jnp.dot(oh_ref[0], tab_ref[...], preferred_element_type=F32)


def _t5(onehot, table):
    v, b, nv = onehot.shape
    return pl.pallas_call(
        _t5_body,
        grid=(v,),
        in_specs=[
            pl.BlockSpec((1, b, nv), lambda i: (i, 0, 0)),
            pl.BlockSpec((nv, D), lambda i: (0, 0)),
        ],
        out_specs=pl.BlockSpec((1, b, D), lambda i: (i, 0, 0)),
        out_shape=jax.ShapeDtypeStruct((v, b, D), F32),
    )(onehot, table)


def kernel(dxseqs, drugseqs, dx_onehot, drug_onehot, dxLeavesList,
           dxAncestorsList, drugLeavesList, drugAncestorsList,
           ctd_dx_leaves_list, ctd_dx_ancesster_list, ctd_dx_rel_list,
           ctd_dx_permute_list, ctd_rx_leaves_list, ctd_rx_ancesster_list,
           ctd_rx_rel_list, ctd_rx_permute_list, EHRdxEmb_W, EHRdrugEmb_W,
           dxOntoW, drugOntoW, dxAttnW, dxAttnb, dxAttnU, drugAttnW,
           drugAttnb, drugAttnU, cooccurW, cooccurB):
    B, V, NDX = dxseqs.shape
    NRX = drugseqs.shape[2]
    DXV, MAXA = dxLeavesList.shape
    RXV = drugLeavesList.shape[0]
    AD = dxAttnW.shape[1]

    def padw(m):
        return jnp.pad(m, ((0, 0), (0, ADP - AD)))

    dxa1, dxa2 = padw(dxAttnW[:D]), padw(dxAttnW[D:])
    rxa1, rxa2 = padw(drugAttnW[:D]), padw(drugAttnW[D:])
    dxb = jnp.pad(dxAttnb, (0, ADP - AD)).reshape(1, ADP)
    rxb = jnp.pad(drugAttnb, (0, ADP - AD)).reshape(1, ADP)
    dxu = jnp.pad(dxAttnU[:, 0], (0, ADP - AD))
    rxu = jnp.pad(drugAttnU[:, 0], (0, ADP - AD))

    p1dx, catdx, p1rx, catrx = _t1(dxOntoW, drugOntoW, dxa1, dxa2, dxb,
                                   rxa1, rxa2, rxb)

    def flatpad(ix, n, ch):
        f = ix.reshape(-1).astype(jnp.int32)
        return jnp.pad(f, (0, n - f.shape[0])).reshape(NW, -1, ch)

    npair_dx = _pad_rows(DXV * MAXA, 80)
    npair_rx = _pad_rows(RXV * MAXA, 80)
    g1dx, gcatdx, g1rx, gcatrx = _sc_gathers(
        [p1dx, catdx, p1rx, catrx],
        [flatpad(dxLeavesList, npair_dx, 80), flatpad(dxAncestorsList, npair_dx, 80),
         flatpad(drugLeavesList, npair_rx, 80), flatpad(drugAncestorsList, npair_rx, 80)],
        ch=80,
    )

    dxall = _t2(g1dx.reshape(-1, ADP), gcatdx.reshape(-1, CATW), dxu,
                MAXA, DXV)
    rxall = _t2(g1rx.reshape(-1, ADP), gcatrx.reshape(-1, CATW), rxu,
                MAXA, RXV)

    nseq_dx = _pad_rows(B * V * NDX, 80)
    nseq_rx = _pad_rows(B * V * NRX, 80)
    seqdx_ix = flatpad(dxseqs, nseq_dx, 80)
    seqrx_ix = flatpad(drugseqs, nseq_rx, 80)
    gehr_dx, gall_dx, gehr_rx, gall_rx = _sc_gathers(
        [EHRdxEmb_W, dxall, EHRdrugEmb_W, rxall],
        [seqdx_ix, seqdx_ix, seqrx_ix, seqrx_ix],
        ch=80,
    )

    EHRVEmb = _t3(gehr_dx.reshape(-1, D)[:B * V * NDX],
                  gehr_rx.reshape(-1, D)[:B * V * NRX], NDX, NRX, B * V)

    cooccurU = _t4(EHRVEmb, cooccurW, cooccurB).reshape(B, V, -1)

    dxontoV = _t5(dx_onehot, dxall[:DXV])
    rxontoV = _t5(drug_onehot, rxall[:RXV])

    ontoEmb = jnp.concatenate(
        [gall_dx.reshape(-1, D)[:B * V * NDX].reshape(B, V, NDX, D),
         gall_rx.reshape(-1, D)[:B * V * NRX].reshape(B, V, NRX, D)], axis=2)

    return (cooccurU,
            EHRVEmb.reshape(B, V, D),
            ontoEmb,
            jnp.transpose(dxontoV, (1, 0, 2)),
            jnp.transpose(rxontoV, (1, 0, 2)))


# speedup vs baseline: 1.3295x; 1.0268x over previous
import functools
import jax
import jax.numpy as jnp
from jax import lax
from jax.experimental import pallas as pl
from jax.experimental.pallas import tpu as pltpu
from jax.experimental.pallas import tpu_sc as plsc

F32 = jnp.float32
D = 128
ADP = 128
CATW = D + ADP
NW = 32
def _pad_rows(n, ch):
    q = NW * ch
    return ((n + q - 1) // q) * q


def _t1_body(wdx_ref, wrx_ref, dxa1_ref, dxa2_ref, dxb_ref,
             rxa1_ref, rxa2_ref, rxb_ref,
             p1dx_ref, catdx_ref, p1rx_ref, catrx_ref):
    wdx = wdx_ref[...]
    p1dx_ref[...] = jnp.dot(wdx, dxa1_ref[...], preferred_element_type=F32)
    catdx_ref[:, :D] = wdx
    catdx_ref[:, D:] = jnp.dot(wdx, dxa2_ref[...], preferred_element_type=F32) + dxb_ref[...]
    wrx = wrx_ref[...]
    p1rx_ref[...] = jnp.dot(wrx, rxa1_ref[...], preferred_element_type=F32)
    catrx_ref[:, :D] = wrx
    catrx_ref[:, D:] = jnp.dot(wrx, rxa2_ref[...], preferred_element_type=F32) + rxb_ref[...]


def _t1(wdx, wrx, dxa1, dxa2, dxb, rxa1, rxa2, rxb):
    vdx, vrx = wdx.shape[0], wrx.shape[0]
    return pl.pallas_call(
        _t1_body,
        out_shape=[
            jax.ShapeDtypeStruct((vdx, ADP), F32),
            jax.ShapeDtypeStruct((vdx, CATW), F32),
            jax.ShapeDtypeStruct((vrx, ADP), F32),
            jax.ShapeDtypeStruct((vrx, CATW), F32),
        ],
    )(wdx, wrx, dxa1, dxa2, dxb, rxa1, rxa2, rxb)


def _run_spec(tab, idx3d, out, idxbuf, rows, gsem, wsem, wid, k, ch):
    base = wid * k
    pltpu.sync_copy(idx3d.at[wid], idxbuf)

    def wait(par, sem):
        pltpu.make_async_copy(out.at[0], rows.at[par], sem.at[par]).wait()

    pltpu.async_copy(tab.at[idxbuf.at[0]], rows.at[0], gsem.at[0])

    def body(c, carry):
        par = lax.rem(c, 2)
        prv = lax.rem(c + 1, 2)

        @pl.when(c >= 2)
        def _():
            wait(par, wsem)

        pltpu.async_copy(tab.at[idxbuf.at[c]], rows.at[par], gsem.at[par])
        wait(prv, gsem)
        pltpu.async_copy(rows.at[prv], out.at[base + c - 1], wsem.at[prv])
        return carry

    lax.fori_loop(1, k, body, 0)
    last = (k - 1) % 2
    wait(last, gsem)
    pltpu.async_copy(rows.at[last], out.at[base + k - 1], wsem.at[last])
    wait((k - 2) % 2, wsem)
    wait(last, wsem)


def _sc_gather_body(widths, chunks, ch, *refs):
    n = len(widths)
    tabs = refs[:n]
    idxs = refs[n:2 * n]
    outs = refs[2 * n:3 * n]
    scratch = list(refs[3 * n:])
    wid = lax.axis_index("s") * 2 + lax.axis_index("c")
    idxbufs = scratch[:n]
    per_w = {}
    for w in sorted(set(widths)):
        per_w[w] = tuple(scratch[n:n + 3])
        del scratch[n:n + 3]
    for i in range(n):
        rows, gsem, wsem = per_w[widths[i]]
        _run_spec(tabs[i], idxs[i], outs[i], idxbufs[i], rows, gsem, wsem,
                  wid, chunks[i], ch)


def _sc_gathers(tables, idx_lists, ch):
    widths = tuple(int(t.shape[1]) for t in tables)
    chunks = tuple(int(ix.shape[1]) for ix in idx_lists)
    scratch = [pltpu.VMEM((k, ch), jnp.int32) for k in chunks]
    for w in sorted(set(widths)):
        scratch.append(pltpu.VMEM((2, ch, w), F32))
        scratch.append(pltpu.SemaphoreType.DMA((2,)))
        scratch.append(pltpu.SemaphoreType.DMA((2,)))
    out_type = [jax.ShapeDtypeStruct((NW * k, ch, w), F32)
                for k, w in zip(chunks, widths)]
    mesh = plsc.VectorSubcoreMesh(core_axis_name="c", subcore_axis_name="s")
    k = pl.kernel(
        functools.partial(_sc_gather_body, widths, chunks, ch),
        out_type=out_type,
        mesh=mesh,
        scratch_types=scratch,
    )
    return k(*tables, *idx_lists)


def _t2_body(nvalid, lb, g1_ref, cat_ref, u_ref, out_ref):
    g1 = g1_ref[...]
    cat = cat_ref[...]
    ea = cat[:, :, :D]
    p2 = cat[:, :, D:]
    t = jnp.tanh(g1 + p2)
    pre = jnp.sum(t * u_ref[...], axis=2)
    m = jnp.max(pre, axis=1, keepdims=True)
    e = jnp.exp(pre - m)
    attn = e / jnp.sum(e, axis=1, keepdims=True)
    res = jnp.sum(attn[:, :, None] * ea, axis=1)
    rid = pl.program_id(0) * lb + lax.broadcasted_iota(jnp.int32, (lb, 1), 0)
    out_ref[...] = jnp.where(rid < nvalid, res, 0.0)


def _t2(g1, cat, u, maxa, nvalid, lb=256):
    nl = g1.shape[0] // maxa
    g1 = g1.reshape(nl, maxa, ADP)
    cat = cat.reshape(nl, maxa, CATW)
    grid = nl // lb
    return pl.pallas_call(
        functools.partial(_t2_body, nvalid, lb),
        grid=(grid,),
        in_specs=[
            pl.BlockSpec((lb, maxa, ADP), lambda i: (i, 0, 0)),
            pl.BlockSpec((lb, maxa, CATW), lambda i: (i, 0, 0)),
            pl.BlockSpec((1, 1, ADP), lambda i: (0, 0, 0)),
        ],
        out_specs=pl.BlockSpec((lb, D), lambda i: (i, 0)),
        out_shape=jax.ShapeDtypeStruct((nl, D), F32),
    )(g1, cat, u.reshape(1, 1, ADP))


def _t3_body(ndx, dxe_ref, rxe_ref, dxa_ref, rxa_ref, ehr_ref, onto_ref):
    sdx = jnp.sum(dxe_ref[...], axis=1)
    srx = jnp.sum(rxe_ref[...], axis=1)
    vdx = sdx * lax.rsqrt(jnp.maximum(jnp.sum(sdx * sdx, axis=1, keepdims=True), 1e-24))
    vrx = srx * lax.rsqrt(jnp.maximum(jnp.sum(srx * srx, axis=1, keepdims=True), 1e-24))
    ehr_ref[...] = vdx + vrx
    onto_ref[:, :ndx, :] = dxa_ref[...]
    onto_ref[:, ndx:, :] = rxa_ref[...]


def _t3(gedx, gerx, gadx, garx, ndx, nrx, rows, rb=64):
    gedx = gedx.reshape(rows, ndx, D)
    gerx = gerx.reshape(rows, nrx, D)
    gadx = gadx.reshape(rows, ndx, D)
    garx = garx.reshape(rows, nrx, D)
    nt = ndx + nrx
    return pl.pallas_call(
        functools.partial(_t3_body, ndx),
        grid=(rows // rb,),
        in_specs=[
            pl.BlockSpec((rb, ndx, D), lambda i: (i, 0, 0)),
            pl.BlockSpec((rb, nrx, D), lambda i: (i, 0, 0)),
            pl.BlockSpec((rb, ndx, D), lambda i: (i, 0, 0)),
            pl.BlockSpec((rb, nrx, D), lambda i: (i, 0, 0)),
        ],
        out_specs=[
            pl.BlockSpec((rb, D), lambda i: (i, 0)),
            pl.BlockSpec((rb, nt, D), lambda i: (i, 0, 0)),
        ],
        out_shape=[
            jax.ShapeDtypeStruct((rows, D), F32),
            jax.ShapeDtypeStruct((rows, nt, D), F32),
        ],
    )(gedx, gerx, gadx, garx)


def _t4_body(x_ref, w_ref, b_ref, out_ref):
    y = jnp.dot(x_ref[...], w_ref[...], preferred_element_type=F32) + b_ref[...]
    m = jnp.max(y, axis=1, keepdims=True)
    e = jnp.exp(y - m)
    out_ref[...] = e / jnp.sum(e, axis=1, keepdims=True)


def _t4(x, w, b, rb=64):
    rows, nv = x.shape[0], w.shape[1]
    return pl.pallas_call(
        _t4_body,
        grid=(rows // rb,),
        in_specs=[
            pl.BlockSpec((rb, D), lambda i: (i, 0)),
            pl.BlockSpec((D, nv), lambda i: (0, 0)),
            pl.BlockSpec((1, nv), lambda i: (0, 0)),
        ],
        out_specs=pl.BlockSpec((rb, nv), lambda i: (i, 0)),
        out_shape=jax.ShapeDtypeStruct((rows, nv), F32),
    )(x, w, b.reshape(1, nv))


def _t5_body(oh_ref, tab_ref, out_ref):
    out_ref[0] = jnp.dot(oh_ref[0], tab_ref[...], preferred_element_type=F32)


def _t5(onehot, table):
    v, b, nv = onehot.shape
    return pl.pallas_call(
        _t5_body,
        grid=(v,),
        in_specs=[
            pl.BlockSpec((1, b, nv), lambda i: (i, 0, 0)),
            pl.BlockSpec((nv, D), lambda i: (0, 0)),
        ],
        out_specs=pl.BlockSpec((1, b, D), lambda i: (i, 0, 0)),
        out_shape=jax.ShapeDtypeStruct((v, b, D), F32),
    )(onehot, table)


def kernel(dxseqs, drugseqs, dx_onehot, drug_onehot, dxLeavesList,
           dxAncestorsList, drugLeavesList, drugAncestorsList,
           ctd_dx_leaves_list, ctd_dx_ancesster_list, ctd_dx_rel_list,
           ctd_dx_permute_list, ctd_rx_leaves_list, ctd_rx_ancesster_list,
           ctd_rx_rel_list, ctd_rx_permute_list, EHRdxEmb_W, EHRdrugEmb_W,
           dxOntoW, drugOntoW, dxAttnW, dxAttnb, dxAttnU, drugAttnW,
           drugAttnb, drugAttnU, cooccurW, cooccurB):
    B, V, NDX = dxseqs.shape
    NRX = drugseqs.shape[2]
    DXV, MAXA = dxLeavesList.shape
    RXV = drugLeavesList.shape[0]
    AD = dxAttnW.shape[1]

    def padw(m):
        return jnp.pad(m, ((0, 0), (0, ADP - AD)))

    dxa1, dxa2 = padw(dxAttnW[:D]), padw(dxAttnW[D:])
    rxa1, rxa2 = padw(drugAttnW[:D]), padw(drugAttnW[D:])
    dxb = jnp.pad(dxAttnb, (0, ADP - AD)).reshape(1, ADP)
    rxb = jnp.pad(drugAttnb, (0, ADP - AD)).reshape(1, ADP)
    dxu = jnp.pad(dxAttnU[:, 0], (0, ADP - AD))
    rxu = jnp.pad(drugAttnU[:, 0], (0, ADP - AD))

    p1dx, catdx, p1rx, catrx = _t1(dxOntoW, drugOntoW, dxa1, dxa2, dxb,
                                   rxa1, rxa2, rxb)

    def flatpad(ix, n, ch):
        f = ix.reshape(-1).astype(jnp.int32)
        return jnp.pad(f, (0, n - f.shape[0])).reshape(NW, -1, ch)

    npair_dx = _pad_rows(DXV * MAXA, 80)
    npair_rx = _pad_rows(RXV * MAXA, 80)
    nseq_dx = _pad_rows(B * V * NDX, 80)
    nseq_rx = _pad_rows(B * V * NRX, 80)
    seqdx_ix = flatpad(dxseqs, nseq_dx, 80)
    seqrx_ix = flatpad(drugseqs, nseq_rx, 80)
    g1dx, gcatdx, g1rx, gcatrx, gehr_dx, gehr_rx = _sc_gathers(
        [p1dx, catdx, p1rx, catrx, EHRdxEmb_W, EHRdrugEmb_W],
        [flatpad(dxLeavesList, npair_dx, 80), flatpad(dxAncestorsList, npair_dx, 80),
         flatpad(drugLeavesList, npair_rx, 80), flatpad(drugAncestorsList, npair_rx, 80),
         seqdx_ix, seqrx_ix],
        ch=80,
    )

    dxall = _t2(g1dx.reshape(-1, ADP), gcatdx.reshape(-1, CATW), dxu,
                MAXA, DXV)
    rxall = _t2(g1rx.reshape(-1, ADP), gcatrx.reshape(-1, CATW), rxu,
                MAXA, RXV)

    gall_dx, gall_rx = _sc_gathers(
        [dxall, rxall],
        [seqdx_ix, seqrx_ix],
        ch=80,
    )

    EHRVEmb, onto = _t3(gehr_dx.reshape(-1, D)[:B * V * NDX],
                        gehr_rx.reshape(-1, D)[:B * V * NRX],
                        gall_dx.reshape(-1, D)[:B * V * NDX],
                        gall_rx.reshape(-1, D)[:B * V * NRX],
                        NDX, NRX, B * V)

    cooccurU = _t4(EHRVEmb, cooccurW, cooccurB).reshape(B, V, -1)

    dxontoV = _t5(dx_onehot, dxall[:DXV])
    rxontoV = _t5(drug_onehot, rxall[:RXV])

    ontoEmb = onto.reshape(B, V, NDX + NRX, D)

    return (cooccurU,
            EHRVEmb.reshape(B, V, D),
            ontoEmb,
            jnp.transpose(dxontoV, (1, 0, 2)),
            jnp.transpose(rxontoV, (1, 0, 2)))


# speedup vs baseline: 1.5013x; 1.1292x over previous
import functools
import jax
import jax.numpy as jnp
from jax import lax
from jax.experimental import pallas as pl
from jax.experimental.pallas import tpu as pltpu
from jax.experimental.pallas import tpu_sc as plsc

F32 = jnp.float32
D = 128
ADP = 128
NW = 32
def _pad_rows(n, ch):
    q = NW * ch
    return ((n + q - 1) // q) * q


def _bf16_bits(x):
    b = lax.bitcast_convert_type(x, jnp.int32)
    r = b + 0x7FFF + ((b >> 16) & 1)
    return (r >> 16) & 0xFFFF


def _pack2(lo, hi):
    return _bf16_bits(lo) | (_bf16_bits(hi) << 16)


def _t1_body(wdx_ref, wrx_ref, dxa1_ref, dxa2_ref, dxb_ref,
             rxa1_ref, rxa2_ref, rxb_ref,
             p1dx_ref, padx_ref, p1rx_ref, parx_ref):
    wdx = wdx_ref[...]
    p1dx_ref[...] = jnp.dot(wdx, dxa1_ref[...], preferred_element_type=F32)
    p2dx = jnp.dot(wdx, dxa2_ref[...], preferred_element_type=F32) + dxb_ref[...]
    padx_ref[...] = _pack2(wdx, p2dx)
    wrx = wrx_ref[...]
    p1rx_ref[...] = jnp.dot(wrx, rxa1_ref[...], preferred_element_type=F32)
    p2rx = jnp.dot(wrx, rxa2_ref[...], preferred_element_type=F32) + rxb_ref[...]
    parx_ref[...] = _pack2(wrx, p2rx)


def _t1(wdx, wrx, dxa1, dxa2, dxb, rxa1, rxa2, rxb):
    vdx, vrx = wdx.shape[0], wrx.shape[0]
    return pl.pallas_call(
        _t1_body,
        out_shape=[
            jax.ShapeDtypeStruct((vdx, ADP), F32),
            jax.ShapeDtypeStruct((vdx, D), jnp.int32),
            jax.ShapeDtypeStruct((vrx, ADP), F32),
            jax.ShapeDtypeStruct((vrx, D), jnp.int32),
        ],
    )(wdx, wrx, dxa1, dxa2, dxb, rxa1, rxa2, rxb)


def _run_spec(tab, idx3d, out, idxbuf, rows, gsem, wsem, wid, k, ch):
    base = wid * k
    pltpu.sync_copy(idx3d.at[wid], idxbuf)

    def wait(par, sem):
        pltpu.make_async_copy(out.at[0], rows.at[par], sem.at[par]).wait()

    pltpu.async_copy(tab.at[idxbuf.at[0]], rows.at[0], gsem.at[0])

    def body(c, carry):
        par = lax.rem(c, 2)
        prv = lax.rem(c + 1, 2)

        @pl.when(c >= 2)
        def _():
            wait(par, wsem)

        pltpu.async_copy(tab.at[idxbuf.at[c]], rows.at[par], gsem.at[par])
        wait(prv, gsem)
        pltpu.async_copy(rows.at[prv], out.at[base + c - 1], wsem.at[prv])
        return carry

    lax.fori_loop(1, k, body, 0)
    last = (k - 1) % 2
    wait(last, gsem)
    pltpu.async_copy(rows.at[last], out.at[base + k - 1], wsem.at[last])
    wait((k - 2) % 2, wsem)
    wait(last, wsem)


def _sc_gather_body(keys, chunks, ch, *refs):
    n = len(keys)
    tabs = refs[:n]
    idxs = refs[n:2 * n]
    outs = refs[2 * n:3 * n]
    scratch = list(refs[3 * n:])
    wid = lax.axis_index("s") * 2 + lax.axis_index("c")
    idxbufs = scratch[:n]
    per_k = {}
    for kk in sorted(set(keys)):
        per_k[kk] = tuple(scratch[n:n + 3])
        del scratch[n:n + 3]
    for i in range(n):
        rows, gsem, wsem = per_k[keys[i]]
        _run_spec(tabs[i], idxs[i], outs[i], idxbufs[i], rows, gsem, wsem,
                  wid, chunks[i], ch)


def _sc_gathers(tables, idx_lists, ch):
    keys = tuple((int(t.shape[1]), str(t.dtype)) for t in tables)
    chunks = tuple(int(ix.shape[1]) for ix in idx_lists)
    scratch = [pltpu.VMEM((k, ch), jnp.int32) for k in chunks]
    for w, dt in sorted(set(keys)):
        scratch.append(pltpu.VMEM((2, ch, w), jnp.dtype(dt)))
        scratch.append(pltpu.SemaphoreType.DMA((2,)))
        scratch.append(pltpu.SemaphoreType.DMA((2,)))
    out_type = [jax.ShapeDtypeStruct((NW * k, ch, w), jnp.dtype(dt))
                for k, (w, dt) in zip(chunks, keys)]
    mesh = plsc.VectorSubcoreMesh(core_axis_name="c", subcore_axis_name="s")
    k = pl.kernel(
        functools.partial(_sc_gather_body, keys, chunks, ch),
        out_type=out_type,
        mesh=mesh,
        scratch_types=scratch,
    )
    return k(*tables, *idx_lists)


def _t2_body(nvalid, lb, g1_ref, pa_ref, u_ref, out_ref):
    g1 = g1_ref[...]
    pa = pa_ref[...]
    ea = lax.bitcast_convert_type(pa << 16, F32)
    p2 = lax.bitcast_convert_type(pa & jnp.int32(-65536), F32)
    t = jnp.tanh(g1 + p2)
    pre = jnp.sum(t * u_ref[...], axis=2)
    m = jnp.max(pre, axis=1, keepdims=True)
    e = jnp.exp(pre - m)
    attn = e / jnp.sum(e, axis=1, keepdims=True)
    res = jnp.sum(attn[:, :, None] * ea, axis=1)
    rid = pl.program_id(0) * lb + lax.broadcasted_iota(jnp.int32, (lb, 1), 0)
    out_ref[...] = jnp.where(rid < nvalid, res, 0.0)


def _t2(g1, pa, u, maxa, nvalid, lb=256):
    nl = g1.shape[0] // maxa
    g1 = g1.reshape(nl, maxa, ADP)
    pa = pa.reshape(nl, maxa, D)
    grid = nl // lb
    return pl.pallas_call(
        functools.partial(_t2_body, nvalid, lb),
        grid=(grid,),
        in_specs=[
            pl.BlockSpec((lb, maxa, ADP), lambda i: (i, 0, 0)),
            pl.BlockSpec((lb, maxa, D), lambda i: (i, 0, 0)),
            pl.BlockSpec((1, 1, ADP), lambda i: (0, 0, 0)),
        ],
        out_specs=pl.BlockSpec((lb, D), lambda i: (i, 0)),
        out_shape=jax.ShapeDtypeStruct((nl, D), F32),
    )(g1, pa, u.reshape(1, 1, ADP))


def _t3_body(ndx, dxe_ref, rxe_ref, dxa_ref, rxa_ref, ehr_ref, onto_ref):
    sdx = jnp.sum(dxe_ref[...], axis=1)
    srx = jnp.sum(rxe_ref[...], axis=1)
    vdx = sdx * lax.rsqrt(jnp.maximum(jnp.sum(sdx * sdx, axis=1, keepdims=True), 1e-24))
    vrx = srx * lax.rsqrt(jnp.maximum(jnp.sum(srx * srx, axis=1, keepdims=True), 1e-24))
    ehr_ref[...] = vdx + vrx
    onto_ref[:, :ndx, :] = dxa_ref[...]
    onto_ref[:, ndx:, :] = rxa_ref[...]


def _t3(gedx, gerx, gadx, garx, ndx, nrx, rows, rb=64):
    gedx = gedx.reshape(rows, ndx, D)
    gerx = gerx.reshape(rows, nrx, D)
    gadx = gadx.reshape(rows, ndx, D)
    garx = garx.reshape(rows, nrx, D)
    nt = ndx + nrx
    return pl.pallas_call(
        functools.partial(_t3_body, ndx),
        grid=(rows // rb,),
        in_specs=[
            pl.BlockSpec((rb, ndx, D), lambda i: (i, 0, 0)),
            pl.BlockSpec((rb, nrx, D), lambda i: (i, 0, 0)),
            pl.BlockSpec((rb, ndx, D), lambda i: (i, 0, 0)),
            pl.BlockSpec((rb, nrx, D), lambda i: (i, 0, 0)),
        ],
        out_specs=[
            pl.BlockSpec((rb, D), lambda i: (i, 0)),
            pl.BlockSpec((rb, nt, D), lambda i: (i, 0, 0)),
        ],
        out_shape=[
            jax.ShapeDtypeStruct((rows, D), F32),
            jax.ShapeDtypeStruct((rows, nt, D), F32),
        ],
    )(gedx, gerx, gadx, garx)


def _t4_body(x_ref, w_ref, b_ref, out_ref):
    y = jnp.dot(x_ref[...], w_ref[...], preferred_element_type=F32) + b_ref[...]
    m = jnp.max(y, axis=1, keepdims=True)
    e = jnp.exp(y - m)
    out_ref[...] = e / jnp.sum(e, axis=1, keepdims=True)


def _t4(x, w, b, rb=64):
    rows, nv = x.shape[0], w.shape[1]
    return pl.pallas_call(
        _t4_body,
        grid=(rows // rb,),
        in_specs=[
            pl.BlockSpec((rb, D), lambda i: (i, 0)),
            pl.BlockSpec((D, nv), lambda i: (0, 0)),
            pl.BlockSpec((1, nv), lambda i: (0, 0)),
        ],
        out_specs=pl.BlockSpec((rb, nv), lambda i: (i, 0)),
        out_shape=jax.ShapeDtypeStruct((rows, nv), F32),
    )(x, w, b.reshape(1, nv))


def _t5_body(oh_ref, tab_ref, out_ref):
    out_ref[0] = jnp.dot(oh_ref[0], tab_ref[...], preferred_element_type=F32)


def _t5(onehot, table):
    v, b, nv = onehot.shape
    return pl.pallas_call(
        _t5_body,
        grid=(v,),
        in_specs=[
            pl.BlockSpec((1, b, nv), lambda i: (i, 0, 0)),
            pl.BlockSpec((nv, D), lambda i: (0, 0)),
        ],
        out_specs=pl.BlockSpec((1, b, D), lambda i: (i, 0, 0)),
        out_shape=jax.ShapeDtypeStruct((v, b, D), F32),
    )(onehot, table)


def kernel(dxseqs, drugseqs, dx_onehot, drug_onehot, dxLeavesList,
           dxAncestorsList, drugLeavesList, drugAncestorsList,
           ctd_dx_leaves_list, ctd_dx_ancesster_list, ctd_dx_rel_list,
           ctd_dx_permute_list, ctd_rx_leaves_list, ctd_rx_ancesster_list,
           ctd_rx_rel_list, ctd_rx_permute_list, EHRdxEmb_W, EHRdrugEmb_W,
           dxOntoW, drugOntoW, dxAttnW, dxAttnb, dxAttnU, drugAttnW,
           drugAttnb, drugAttnU, cooccurW, cooccurB):
    B, V, NDX = dxseqs.shape
    NRX = drugseqs.shape[2]
    DXV, MAXA = dxLeavesList.shape
    RXV = drugLeavesList.shape[0]
    AD = dxAttnW.shape[1]

    def padw(m):
        return jnp.pad(m, ((0, 0), (0, ADP - AD)))

    dxa1, dxa2 = padw(dxAttnW[:D]), padw(dxAttnW[D:])
    rxa1, rxa2 = padw(drugAttnW[:D]), padw(drugAttnW[D:])
    dxb = jnp.pad(dxAttnb, (0, ADP - AD)).reshape(1, ADP)
    rxb = jnp.pad(drugAttnb, (0, ADP - AD)).reshape(1, ADP)
    dxu = jnp.pad(dxAttnU[:, 0], (0, ADP - AD))
    rxu = jnp.pad(drugAttnU[:, 0], (0, ADP - AD))

    p1dx, padx, p1rx, parx = _t1(dxOntoW, drugOntoW, dxa1, dxa2, dxb,
                                 rxa1, rxa2, rxb)

    def flatpad(ix, n, ch):
        f = ix.reshape(-1).astype(jnp.int32)
        return jnp.pad(f, (0, n - f.shape[0])).reshape(NW, -1, ch)

    npair_dx = _pad_rows(DXV * MAXA, 80)
    npair_rx = _pad_rows(RXV * MAXA, 80)
    nseq_dx = _pad_rows(B * V * NDX, 80)
    nseq_rx = _pad_rows(B * V * NRX, 80)
    seqdx_ix = flatpad(dxseqs, nseq_dx, 80)
    seqrx_ix = flatpad(drugseqs, nseq_rx, 80)
    g1dx, gpadx, g1rx, gparx, gehr_dx, gehr_rx = _sc_gathers(
        [p1dx, padx, p1rx, parx, EHRdxEmb_W, EHRdrugEmb_W],
        [flatpad(dxLeavesList, npair_dx, 80), flatpad(dxAncestorsList, npair_dx, 80),
         flatpad(drugLeavesList, npair_rx, 80), flatpad(drugAncestorsList, npair_rx, 80),
         seqdx_ix, seqrx_ix],
        ch=80,
    )

    dxall = _t2(g1dx.reshape(-1, ADP), gpadx.reshape(-1, D), dxu,
                MAXA, DXV)
    rxall = _t2(g1rx.reshape(-1, ADP), gparx.reshape(-1, D), rxu,
                MAXA, RXV)

    gall_dx, gall_rx = _sc_gathers(
        [dxall, rxall],
        [seqdx_ix, seqrx_ix],
        ch=80,
    )

    EHRVEmb, onto = _t3(gehr_dx.reshape(-1, D)[:B * V * NDX],
                        gehr_rx.reshape(-1, D)[:B * V * NRX],
                        gall_dx.reshape(-1, D)[:B * V * NDX],
                        gall_rx.reshape(-1, D)[:B * V * NRX],
                        NDX, NRX, B * V)

    cooccurU = _t4(EHRVEmb, cooccurW, cooccurB).reshape(B, V, -1)

    dxontoV = _t5(dx_onehot, dxall[:DXV])
    rxontoV = _t5(drug_onehot, rxall[:RXV])

    ontoEmb = onto.reshape(B, V, NDX + NRX, D)

    return (cooccurU,
            EHRVEmb.reshape(B, V, D),
            ontoEmb,
            jnp.transpose(dxontoV, (1, 0, 2)),
            jnp.transpose(rxontoV, (1, 0, 2)))


# speedup vs baseline: 1.5736x; 1.0481x over previous
import functools
import jax
import jax.numpy as jnp
from jax import lax
from jax.experimental import pallas as pl
from jax.experimental.pallas import tpu as pltpu
from jax.experimental.pallas import tpu_sc as plsc

F32 = jnp.float32
D = 128
ADP = 128
NW = 32
def _pad_rows(n, ch):
    q = NW * ch
    return ((n + q - 1) // q) * q


def _bf16_bits(x):
    b = lax.bitcast_convert_type(x, jnp.int32)
    r = b + 0x7FFF + ((b >> 16) & 1)
    return (r >> 16) & 0xFFFF


def _pack2(lo, hi):
    return _bf16_bits(lo) | (_bf16_bits(hi) << 16)


def _t1_body(wdx_ref, wrx_ref, dxa1_ref, dxa2_ref, dxb_ref,
             rxa1_ref, rxa2_ref, rxb_ref,
             p1dx_ref, padx_ref, p1rx_ref, parx_ref):
    wdx = wdx_ref[...]
    p1dx_ref[...] = jnp.dot(wdx, dxa1_ref[...], preferred_element_type=F32)
    p2dx = jnp.dot(wdx, dxa2_ref[...], preferred_element_type=F32) + dxb_ref[...]
    padx_ref[...] = _pack2(wdx, p2dx)
    wrx = wrx_ref[...]
    p1rx_ref[...] = jnp.dot(wrx, rxa1_ref[...], preferred_element_type=F32)
    p2rx = jnp.dot(wrx, rxa2_ref[...], preferred_element_type=F32) + rxb_ref[...]
    parx_ref[...] = _pack2(wrx, p2rx)


def _t1(wdx, wrx, dxa1, dxa2, dxb, rxa1, rxa2, rxb):
    vdx, vrx = wdx.shape[0], wrx.shape[0]
    return pl.pallas_call(
        _t1_body,
        out_shape=[
            jax.ShapeDtypeStruct((vdx, ADP), F32),
            jax.ShapeDtypeStruct((vdx, D), jnp.int32),
            jax.ShapeDtypeStruct((vrx, ADP), F32),
            jax.ShapeDtypeStruct((vrx, D), jnp.int32),
        ],
    )(wdx, wrx, dxa1, dxa2, dxb, rxa1, rxa2, rxb)


NBUF = 4


def _run_spec(tab, idx3d, out, idxbuf, rows, gsem, wsem, wid, k, ch):
    base = wid * k
    pltpu.sync_copy(idx3d.at[wid], idxbuf)

    def wait(par, sem):
        pltpu.make_async_copy(out.at[0], rows.at[par], sem.at[par]).wait()

    for i in range(min(NBUF, k)):
        pltpu.async_copy(tab.at[idxbuf.at[i]], rows.at[i], gsem.at[i])

    def body(c, carry):
        par = lax.rem(c, NBUF)
        wait(par, gsem)
        pltpu.async_copy(rows.at[par], out.at[base + c], wsem.at[par])

        @pl.when(c + NBUF < k)
        def _():
            wait(par, wsem)
            pltpu.async_copy(tab.at[idxbuf.at[c + NBUF]], rows.at[par],
                             gsem.at[par])
        return carry

    lax.fori_loop(0, k, body, 0)
    for c in range(max(0, k - NBUF), k):
        wait(c % NBUF, wsem)


def _sc_gather_body(keys, chunks, ch, *refs):
    n = len(keys)
    tabs = refs[:n]
    idxs = refs[n:2 * n]
    outs = refs[2 * n:3 * n]
    scratch = list(refs[3 * n:])
    wid = lax.axis_index("s") * 2 + lax.axis_index("c")
    idxbufs = scratch[:n]
    per_k = {}
    for kk in sorted(set(keys)):
        per_k[kk] = tuple(scratch[n:n + 3])
        del scratch[n:n + 3]
    for i in range(n):
        rows, gsem, wsem = per_k[keys[i]]
        _run_spec(tabs[i], idxs[i], outs[i], idxbufs[i], rows, gsem, wsem,
                  wid, chunks[i], ch)


def _sc_gathers(tables, idx_lists, ch):
    keys = tuple((int(t.shape[1]), str(t.dtype)) for t in tables)
    chunks = tuple(int(ix.shape[1]) for ix in idx_lists)
    scratch = [pltpu.VMEM((k, ch), jnp.int32) for k in chunks]
    for w, dt in sorted(set(keys)):
        scratch.append(pltpu.VMEM((NBUF, ch, w), jnp.dtype(dt)))
        scratch.append(pltpu.SemaphoreType.DMA((NBUF,)))
        scratch.append(pltpu.SemaphoreType.DMA((NBUF,)))
    out_type = [jax.ShapeDtypeStruct((NW * k, ch, w), jnp.dtype(dt))
                for k, (w, dt) in zip(chunks, keys)]
    mesh = plsc.VectorSubcoreMesh(core_axis_name="c", subcore_axis_name="s")
    k = pl.kernel(
        functools.partial(_sc_gather_body, keys, chunks, ch),
        out_type=out_type,
        mesh=mesh,
        scratch_types=scratch,
    )
    return k(*tables, *idx_lists)


def _t2_body(nvalid, lb, g1_ref, pa_ref, u_ref, out_ref):
    g1 = g1_ref[...]
    pa = pa_ref[...]
    ea = lax.bitcast_convert_type(pa << 16, F32)
    p2 = lax.bitcast_convert_type(pa & jnp.int32(-65536), F32)
    t = jnp.tanh(g1 + p2)
    pre = jnp.sum(t * u_ref[...], axis=2)
    m = jnp.max(pre, axis=1, keepdims=True)
    e = jnp.exp(pre - m)
    attn = e / jnp.sum(e, axis=1, keepdims=True)
    res = jnp.sum(attn[:, :, None] * ea, axis=1)
    rid = pl.program_id(0) * lb + lax.broadcasted_iota(jnp.int32, (lb, 1), 0)
    out_ref[...] = jnp.where(rid < nvalid, res, 0.0)


def _t2(g1, pa, u, maxa, nvalid, lb=256):
    nl = g1.shape[0] // maxa
    g1 = g1.reshape(nl, maxa, ADP)
    pa = pa.reshape(nl, maxa, D)
    grid = nl // lb
    return pl.pallas_call(
        functools.partial(_t2_body, nvalid, lb),
        grid=(grid,),
        in_specs=[
            pl.BlockSpec((lb, maxa, ADP), lambda i: (i, 0, 0)),
            pl.BlockSpec((lb, maxa, D), lambda i: (i, 0, 0)),
            pl.BlockSpec((1, 1, ADP), lambda i: (0, 0, 0)),
        ],
        out_specs=pl.BlockSpec((lb, D), lambda i: (i, 0)),
        out_shape=jax.ShapeDtypeStruct((nl, D), F32),
    )(g1, pa, u.reshape(1, 1, ADP))


def _t3_body(ndx, dxe_ref, rxe_ref, dxa_ref, rxa_ref, ehr_ref, onto_ref):
    sdx = jnp.sum(dxe_ref[...], axis=1)
    srx = jnp.sum(rxe_ref[...], axis=1)
    vdx = sdx * lax.rsqrt(jnp.maximum(jnp.sum(sdx * sdx, axis=1, keepdims=True), 1e-24))
    vrx = srx * lax.rsqrt(jnp.maximum(jnp.sum(srx * srx, axis=1, keepdims=True), 1e-24))
    ehr_ref[...] = vdx + vrx
    onto_ref[:, :ndx, :] = dxa_ref[...]
    onto_ref[:, ndx:, :] = rxa_ref[...]


def _t3(gedx, gerx, gadx, garx, ndx, nrx, rows, rb=64):
    gedx = gedx.reshape(rows, ndx, D)
    gerx = gerx.reshape(rows, nrx, D)
    gadx = gadx.reshape(rows, ndx, D)
    garx = garx.reshape(rows, nrx, D)
    nt = ndx + nrx
    return pl.pallas_call(
        functools.partial(_t3_body, ndx),
        grid=(rows // rb,),
        in_specs=[
            pl.BlockSpec((rb, ndx, D), lambda i: (i, 0, 0)),
            pl.BlockSpec((rb, nrx, D), lambda i: (i, 0, 0)),
            pl.BlockSpec((rb, ndx, D), lambda i: (i, 0, 0)),
            pl.BlockSpec((rb, nrx, D), lambda i: (i, 0, 0)),
        ],
        out_specs=[
            pl.BlockSpec((rb, D), lambda i: (i, 0)),
            pl.BlockSpec((rb, nt, D), lambda i: (i, 0, 0)),
        ],
        out_shape=[
            jax.ShapeDtypeStruct((rows, D), F32),
            jax.ShapeDtypeStruct((rows, nt, D), F32),
        ],
    )(gedx, gerx, gadx, garx)


def _t4_body(x_ref, w_ref, b_ref, out_ref):
    y = jnp.dot(x_ref[...], w_ref[...], preferred_element_type=F32) + b_ref[...]
    m = jnp.max(y, axis=1, keepdims=True)
    e = jnp.exp(y - m)
    out_ref[...] = e / jnp.sum(e, axis=1, keepdims=True)


def _t4(x, w, b, rb=64):
    rows, nv = x.shape[0], w.shape[1]
    return pl.pallas_call(
        _t4_body,
        grid=(rows // rb,),
        in_specs=[
            pl.BlockSpec((rb, D), lambda i: (i, 0)),
            pl.BlockSpec((D, nv), lambda i: (0, 0)),
            pl.BlockSpec((1, nv), lambda i: (0, 0)),
        ],
        out_specs=pl.BlockSpec((rb, nv), lambda i: (i, 0)),
        out_shape=jax.ShapeDtypeStruct((rows, nv), F32),
    )(x, w, b.reshape(1, nv))


def _t5_body(oh_ref, tab_ref, out_ref):
    out_ref[0] = jnp.dot(oh_ref[0], tab_ref[...], preferred_element_type=F32)


def _t5(onehot, table):
    v, b, nv = onehot.shape
    return pl.pallas_call(
        _t5_body,
        grid=(v,),
        in_specs=[
            pl.BlockSpec((1, b, nv), lambda i: (i, 0, 0)),
            pl.BlockSpec((nv, D), lambda i: (0, 0)),
        ],
        out_specs=pl.BlockSpec((1, b, D), lambda i: (i, 0, 0)),
        out_shape=jax.ShapeDtypeStruct((v, b, D), F32),
    )(onehot, table)


def kernel(dxseqs, drugseqs, dx_onehot, drug_onehot, dxLeavesList,
           dxAncestorsList, drugLeavesList, drugAncestorsList,
           ctd_dx_leaves_list, ctd_dx_ancesster_list, ctd_dx_rel_list,
           ctd_dx_permute_list, ctd_rx_leaves_list, ctd_rx_ancesster_list,
           ctd_rx_rel_list, ctd_rx_permute_list, EHRdxEmb_W, EHRdrugEmb_W,
           dxOntoW, drugOntoW, dxAttnW, dxAttnb, dxAttnU, drugAttnW,
           drugAttnb, drugAttnU, cooccurW, cooccurB):
    B, V, NDX = dxseqs.shape
    NRX = drugseqs.shape[2]
    DXV, MAXA = dxLeavesList.shape
    RXV = drugLeavesList.shape[0]
    AD = dxAttnW.shape[1]

    def padw(m):
        return jnp.pad(m, ((0, 0), (0, ADP - AD)))

    dxa1, dxa2 = padw(dxAttnW[:D]), padw(dxAttnW[D:])
    rxa1, rxa2 = padw(drugAttnW[:D]), padw(drugAttnW[D:])
    dxb = jnp.pad(dxAttnb, (0, ADP - AD)).reshape(1, ADP)
    rxb = jnp.pad(drugAttnb, (0, ADP - AD)).reshape(1, ADP)
    dxu = jnp.pad(dxAttnU[:, 0], (0, ADP - AD))
    rxu = jnp.pad(drugAttnU[:, 0], (0, ADP - AD))

    p1dx, padx, p1rx, parx = _t1(dxOntoW, drugOntoW, dxa1, dxa2, dxb,
                                 rxa1, rxa2, rxb)

    def flatpad(ix, n, ch):
        f = ix.reshape(-1).astype(jnp.int32)
        return jnp.pad(f, (0, n - f.shape[0])).reshape(NW, -1, ch)

    npair_dx = _pad_rows(DXV * MAXA, 80)
    npair_rx = _pad_rows(RXV * MAXA, 80)
    nseq_dx = _pad_rows(B * V * NDX, 80)
    nseq_rx = _pad_rows(B * V * NRX, 80)
    seqdx_ix = flatpad(dxseqs, nseq_dx, 80)
    seqrx_ix = flatpad(drugseqs, nseq_rx, 80)
    g1dx, gpadx, g1rx, gparx, gehr_dx, gehr_rx = _sc_gathers(
        [p1dx, padx, p1rx, parx, EHRdxEmb_W, EHRdrugEmb_W],
        [flatpad(dxLeavesList, npair_dx, 80), flatpad(dxAncestorsList, npair_dx, 80),
         flatpad(drugLeavesList, npair_rx, 80), flatpad(drugAncestorsList, npair_rx, 80),
         seqdx_ix, seqrx_ix],
        ch=80,
    )

    dxall = _t2(g1dx.reshape(-1, ADP), gpadx.reshape(-1, D), dxu,
                MAXA, DXV)
    rxall = _t2(g1rx.reshape(-1, ADP), gparx.reshape(-1, D), rxu,
                MAXA, RXV)

    gall_dx, gall_rx = _sc_gathers(
        [dxall, rxall],
        [seqdx_ix, seqrx_ix],
        ch=80,
    )

    EHRVEmb, onto = _t3(gehr_dx.reshape(-1, D)[:B * V * NDX],
                        gehr_rx.reshape(-1, D)[:B * V * NRX],
                        gall_dx.reshape(-1, D)[:B * V * NDX],
                        gall_rx.reshape(-1, D)[:B * V * NRX],
                        NDX, NRX, B * V)

    cooccurU = _t4(EHRVEmb, cooccurW, cooccurB).reshape(B, V, -1)

    dxontoV = _t5(dx_onehot, dxall[:DXV])
    rxontoV = _t5(drug_onehot, rxall[:RXV])

    ontoEmb = onto.reshape(B, V, NDX + NRX, D)

    return (cooccurU,
            EHRVEmb.reshape(B, V, D),
            ontoEmb,
            jnp.transpose(dxontoV, (1, 0, 2)),
            jnp.transpose(rxontoV, (1, 0, 2)))


# speedup vs baseline: 1.5997x; 1.0166x over previous
import functools
import jax
import jax.numpy as jnp
from jax import lax
from jax.experimental import pallas as pl
from jax.experimental.pallas import tpu as pltpu
from jax.experimental.pallas import tpu_sc as plsc

F32 = jnp.float32
D = 128
ADP = 128
NW = 32
def _pad_rows(n, ch):
    q = NW * ch
    return ((n + q - 1) // q) * q


def _bf16_bits(x):
    b = lax.bitcast_convert_type(x, jnp.int32)
    r = b + 0x7FFF + ((b >> 16) & 1)
    return (r >> 16) & 0xFFFF


def _pack2(lo, hi):
    return _bf16_bits(lo) | (_bf16_bits(hi) << 16)


def _t1_body(wdx_ref, wrx_ref, dxa1_ref, dxa2_ref, dxb_ref,
             rxa1_ref, rxa2_ref, rxb_ref,
             p1dx_ref, padx_ref, p1rx_ref, parx_ref):
    wdx = wdx_ref[...]
    p1dx_ref[...] = jnp.dot(wdx, dxa1_ref[...], preferred_element_type=F32)
    p2dx = jnp.dot(wdx, dxa2_ref[...], preferred_element_type=F32) + dxb_ref[...]
    padx_ref[...] = _pack2(wdx, p2dx)
    wrx = wrx_ref[...]
    p1rx_ref[...] = jnp.dot(wrx, rxa1_ref[...], preferred_element_type=F32)
    p2rx = jnp.dot(wrx, rxa2_ref[...], preferred_element_type=F32) + rxb_ref[...]
    parx_ref[...] = _pack2(wrx, p2rx)


def _t1(wdx, wrx, dxa1, dxa2, dxb, rxa1, rxa2, rxb):
    vdx, vrx = wdx.shape[0], wrx.shape[0]
    return pl.pallas_call(
        _t1_body,
        out_shape=[
            jax.ShapeDtypeStruct((vdx, ADP), F32),
            jax.ShapeDtypeStruct((vdx, D), jnp.int32),
            jax.ShapeDtypeStruct((vrx, ADP), F32),
            jax.ShapeDtypeStruct((vrx, D), jnp.int32),
        ],
    )(wdx, wrx, dxa1, dxa2, dxb, rxa1, rxa2, rxb)


NBUF = 4


def _run_spec(tab, idx3d, out, idxbuf, rows, gsem, wsem, wid, k, ch):
    base = wid * k
    pltpu.sync_copy(idx3d.at[wid], idxbuf)

    def wait(par, sem):
        pltpu.make_async_copy(out.at[0], rows.at[par], sem.at[par]).wait()

    for i in range(min(NBUF, k)):
        pltpu.async_copy(tab.at[idxbuf.at[i]], rows.at[i], gsem.at[i])

    def body(c, carry):
        par = lax.rem(c, NBUF)
        wait(par, gsem)
        pltpu.async_copy(rows.at[par], out.at[base + c], wsem.at[par])

        @pl.when(c + NBUF < k)
        def _():
            wait(par, wsem)
            pltpu.async_copy(tab.at[idxbuf.at[c + NBUF]], rows.at[par],
                             gsem.at[par])
        return carry

    lax.fori_loop(0, k, body, 0)
    for c in range(max(0, k - NBUF), k):
        wait(c % NBUF, wsem)


def _sc_gather_body(keys, chunks, ch, *refs):
    n = len(keys)
    tabs = refs[:n]
    idxs = refs[n:2 * n]
    outs = refs[2 * n:3 * n]
    scratch = list(refs[3 * n:])
    wid = lax.axis_index("s") * 2 + lax.axis_index("c")
    idxbufs = scratch[:n]
    per_k = {}
    for kk in sorted(set(keys)):
        per_k[kk] = tuple(scratch[n:n + 3])
        del scratch[n:n + 3]
    for i in range(n):
        rows, gsem, wsem = per_k[keys[i]]
        _run_spec(tabs[i], idxs[i], outs[i], idxbufs[i], rows, gsem, wsem,
                  wid, chunks[i], ch)


def _sc_gathers(tables, idx_lists, ch):
    keys = tuple((int(t.shape[1]), str(t.dtype)) for t in tables)
    chunks = tuple(int(ix.shape[1]) for ix in idx_lists)
    scratch = [pltpu.VMEM((k, ch), jnp.int32) for k in chunks]
    for w, dt in sorted(set(keys)):
        scratch.append(pltpu.VMEM((NBUF, ch, w), jnp.dtype(dt)))
        scratch.append(pltpu.SemaphoreType.DMA((NBUF,)))
        scratch.append(pltpu.SemaphoreType.DMA((NBUF,)))
    out_type = [jax.ShapeDtypeStruct((NW * k, ch, w), jnp.dtype(dt))
                for k, (w, dt) in zip(chunks, keys)]
    mesh = plsc.VectorSubcoreMesh(core_axis_name="c", subcore_axis_name="s")
    k = pl.kernel(
        functools.partial(_sc_gather_body, keys, chunks, ch),
        out_type=out_type,
        mesh=mesh,
        scratch_types=scratch,
    )
    return k(*tables, *idx_lists)


def _t2_body(nvalid, lb, g1_ref, pa_ref, u_ref, ehr_ref, out_ref, pk_ref):
    g1 = g1_ref[...]
    pa = pa_ref[...]
    ea = lax.bitcast_convert_type(pa << 16, F32)
    p2 = lax.bitcast_convert_type(pa & jnp.int32(-65536), F32)
    t = jnp.tanh(g1 + p2)
    pre = jnp.sum(t * u_ref[...], axis=2)
    m = jnp.max(pre, axis=1, keepdims=True)
    e = jnp.exp(pre - m)
    attn = e / jnp.sum(e, axis=1, keepdims=True)
    res = jnp.sum(attn[:, :, None] * ea, axis=1)
    rid = pl.program_id(0) * lb + lax.broadcasted_iota(jnp.int32, (lb, 1), 0)
    res = jnp.where(rid < nvalid, res, 0.0)
    out_ref[...] = res
    pk_ref[...] = _pack2(ehr_ref[...], res)


def _t2(g1, pa, u, ehr, maxa, nvalid, lb=256):
    nl = g1.shape[0] // maxa
    g1 = g1.reshape(nl, maxa, ADP)
    pa = pa.reshape(nl, maxa, D)
    ehr = jnp.pad(ehr, ((0, nl - ehr.shape[0]), (0, 0)))
    grid = nl // lb
    return pl.pallas_call(
        functools.partial(_t2_body, nvalid, lb),
        grid=(grid,),
        in_specs=[
            pl.BlockSpec((lb, maxa, ADP), lambda i: (i, 0, 0)),
            pl.BlockSpec((lb, maxa, D), lambda i: (i, 0, 0)),
            pl.BlockSpec((1, 1, ADP), lambda i: (0, 0, 0)),
            pl.BlockSpec((lb, D), lambda i: (i, 0)),
        ],
        out_specs=[
            pl.BlockSpec((lb, D), lambda i: (i, 0)),
            pl.BlockSpec((lb, D), lambda i: (i, 0)),
        ],
        out_shape=[
            jax.ShapeDtypeStruct((nl, D), F32),
            jax.ShapeDtypeStruct((nl, D), jnp.int32),
        ],
    )(g1, pa, u.reshape(1, 1, ADP), ehr)


def _t3_body(ndx, pdx_ref, prx_ref, ehr_ref, onto_ref):
    pdx = pdx_ref[...]
    prx = prx_ref[...]
    dxe = lax.bitcast_convert_type(pdx << 16, F32)
    rxe = lax.bitcast_convert_type(prx << 16, F32)
    sdx = jnp.sum(dxe, axis=1)
    srx = jnp.sum(rxe, axis=1)
    vdx = sdx * lax.rsqrt(jnp.maximum(jnp.sum(sdx * sdx, axis=1, keepdims=True), 1e-24))
    vrx = srx * lax.rsqrt(jnp.maximum(jnp.sum(srx * srx, axis=1, keepdims=True), 1e-24))
    ehr_ref[...] = vdx + vrx
    onto_ref[:, :ndx, :] = lax.bitcast_convert_type(pdx & jnp.int32(-65536), F32)
    onto_ref[:, ndx:, :] = lax.bitcast_convert_type(prx & jnp.int32(-65536), F32)


def _t3(gpdx, gprx, ndx, nrx, rows, rb=64):
    gpdx = gpdx.reshape(rows, ndx, D)
    gprx = gprx.reshape(rows, nrx, D)
    nt = ndx + nrx
    return pl.pallas_call(
        functools.partial(_t3_body, ndx),
        grid=(rows // rb,),
        in_specs=[
            pl.BlockSpec((rb, ndx, D), lambda i: (i, 0, 0)),
            pl.BlockSpec((rb, nrx, D), lambda i: (i, 0, 0)),
        ],
        out_specs=[
            pl.BlockSpec((rb, D), lambda i: (i, 0)),
            pl.BlockSpec((rb, nt, D), lambda i: (i, 0, 0)),
        ],
        out_shape=[
            jax.ShapeDtypeStruct((rows, D), F32),
            jax.ShapeDtypeStruct((rows, nt, D), F32),
        ],
    )(gpdx, gprx)


def _t4_body(x_ref, w_ref, b_ref, out_ref):
    y = jnp.dot(x_ref[...], w_ref[...], preferred_element_type=F32) + b_ref[...]
    m = jnp.max(y, axis=1, keepdims=True)
    e = jnp.exp(y - m)
    out_ref[...] = e / jnp.sum(e, axis=1, keepdims=True)


def _t4(x, w, b, rb=64):
    rows, nv = x.shape[0], w.shape[1]
    return pl.pallas_call(
        _t4_body,
        grid=(rows // rb,),
        in_specs=[
            pl.BlockSpec((rb, D), lambda i: (i, 0)),
            pl.BlockSpec((D, nv), lambda i: (0, 0)),
            pl.BlockSpec((1, nv), lambda i: (0, 0)),
        ],
        out_specs=pl.BlockSpec((rb, nv), lambda i: (i, 0)),
        out_shape=jax.ShapeDtypeStruct((rows, nv), F32),
    )(x, w, b.reshape(1, nv))


def _t5_body(oh_ref, tab_ref, out_ref):
    out_ref[0] = jnp.dot(oh_ref[0], tab_ref[...], preferred_element_type=F32)


def _t5(onehot, table):
    v, b, nv = onehot.shape
    return pl.pallas_call(
        _t5_body,
        grid=(v,),
        in_specs=[
            pl.BlockSpec((1, b, nv), lambda i: (i, 0, 0)),
            pl.BlockSpec((nv, D), lambda i: (0, 0)),
        ],
        out_specs=pl.BlockSpec((1, b, D), lambda i: (i, 0, 0)),
        out_shape=jax.ShapeDtypeStruct((v, b, D), F32),
    )(onehot, table)


def kernel(dxseqs, drugseqs, dx_onehot, drug_onehot, dxLeavesList,
           dxAncestorsList, drugLeavesList, drugAncestorsList,
           ctd_dx_leaves_list, ctd_dx_ancesster_list, ctd_dx_rel_list,
           ctd_dx_permute_list, ctd_rx_leaves_list, ctd_rx_ancesster_list,
           ctd_rx_rel_list, ctd_rx_permute_list, EHRdxEmb_W, EHRdrugEmb_W,
           dxOntoW, drugOntoW, dxAttnW, dxAttnb, dxAttnU, drugAttnW,
           drugAttnb, drugAttnU, cooccurW, cooccurB):
    B, V, NDX = dxseqs.shape
    NRX = drugseqs.shape[2]
    DXV, MAXA = dxLeavesList.shape
    RXV = drugLeavesList.shape[0]
    AD = dxAttnW.shape[1]

    def padw(m):
        return jnp.pad(m, ((0, 0), (0, ADP - AD)))

    dxa1, dxa2 = padw(dxAttnW[:D]), padw(dxAttnW[D:])
    rxa1, rxa2 = padw(drugAttnW[:D]), padw(drugAttnW[D:])
    dxb = jnp.pad(dxAttnb, (0, ADP - AD)).reshape(1, ADP)
    rxb = jnp.pad(drugAttnb, (0, ADP - AD)).reshape(1, ADP)
    dxu = jnp.pad(dxAttnU[:, 0], (0, ADP - AD))
    rxu = jnp.pad(drugAttnU[:, 0], (0, ADP - AD))

    p1dx, padx, p1rx, parx = _t1(dxOntoW, drugOntoW, dxa1, dxa2, dxb,
                                 rxa1, rxa2, rxb)

    def flatpad(ix, n, ch):
        f = ix.reshape(-1).astype(jnp.int32)
        return jnp.pad(f, (0, n - f.shape[0])).reshape(NW, -1, ch)

    npair_dx = _pad_rows(DXV * MAXA, 80)
    npair_rx = _pad_rows(RXV * MAXA, 80)
    nseq_dx = _pad_rows(B * V * NDX, 80)
    nseq_rx = _pad_rows(B * V * NRX, 80)
    seqdx_ix = flatpad(dxseqs, nseq_dx, 80)
    seqrx_ix = flatpad(drugseqs, nseq_rx, 80)
    g1dx, gpadx, g1rx, gparx = _sc_gathers(
        [p1dx, padx, p1rx, parx],
        [flatpad(dxLeavesList, npair_dx, 80), flatpad(dxAncestorsList, npair_dx, 80),
         flatpad(drugLeavesList, npair_rx, 80), flatpad(drugAncestorsList, npair_rx, 80)],
        ch=80,
    )

    dxall, pkdx = _t2(g1dx.reshape(-1, ADP), gpadx.reshape(-1, D), dxu,
                      EHRdxEmb_W, MAXA, DXV)
    rxall, pkrx = _t2(g1rx.reshape(-1, ADP), gparx.reshape(-1, D), rxu,
                      EHRdrugEmb_W, MAXA, RXV)

    gpk_dx, gpk_rx = _sc_gathers(
        [pkdx, pkrx],
        [seqdx_ix, seqrx_ix],
        ch=80,
    )

    EHRVEmb, onto = _t3(gpk_dx.reshape(-1, D)[:B * V * NDX],
                        gpk_rx.reshape(-1, D)[:B * V * NRX],
                        NDX, NRX, B * V)

    cooccurU = _t4(EHRVEmb, cooccurW, cooccurB).reshape(B, V, -1)

    dxontoV = _t5(dx_onehot, dxall[:DXV])
    rxontoV = _t5(drug_onehot, rxall[:RXV])

    ontoEmb = onto.reshape(B, V, NDX + NRX, D)

    return (cooccurU,
            EHRVEmb.reshape(B, V, D),
            ontoEmb,
            jnp.transpose(dxontoV, (1, 0, 2)),
            jnp.transpose(rxontoV, (1, 0, 2)))


# speedup vs baseline: 1.6124x; 1.0079x over previous
import functools
import jax
import jax.numpy as jnp
from jax import lax
from jax.experimental import pallas as pl
from jax.experimental.pallas import tpu as pltpu
from jax.experimental.pallas import tpu_sc as plsc

F32 = jnp.float32
D = 128
ADP = 128
NW = 32
def _pad_rows(n, ch):
    q = NW * ch
    return ((n + q - 1) // q) * q


def _bf16_bits(x):
    b = lax.bitcast_convert_type(x, jnp.int32)
    r = b + 0x7FFF + ((b >> 16) & 1)
    return (r >> 16) & 0xFFFF


def _pack2(lo, hi):
    return _bf16_bits(lo) | (_bf16_bits(hi) << 16)


def _t1_body(wdx_ref, wrx_ref, dxa1_ref, dxa2_ref, dxb_ref,
             rxa1_ref, rxa2_ref, rxb_ref,
             p1dx_ref, padx_ref, p1rx_ref, parx_ref):
    wdx = wdx_ref[...]
    p1dx_ref[...] = jnp.dot(wdx, dxa1_ref[...], preferred_element_type=F32)
    p2dx = jnp.dot(wdx, dxa2_ref[...], preferred_element_type=F32) + dxb_ref[...]
    padx_ref[...] = _pack2(wdx, p2dx)
    wrx = wrx_ref[...]
    p1rx_ref[...] = jnp.dot(wrx, rxa1_ref[...], preferred_element_type=F32)
    p2rx = jnp.dot(wrx, rxa2_ref[...], preferred_element_type=F32) + rxb_ref[...]
    parx_ref[...] = _pack2(wrx, p2rx)


def _t1(wdx, wrx, dxa1, dxa2, dxb, rxa1, rxa2, rxb):
    vdx, vrx = wdx.shape[0], wrx.shape[0]
    return pl.pallas_call(
        _t1_body,
        out_shape=[
            jax.ShapeDtypeStruct((vdx, ADP), F32),
            jax.ShapeDtypeStruct((vdx, D), jnp.int32),
            jax.ShapeDtypeStruct((vrx, ADP), F32),
            jax.ShapeDtypeStruct((vrx, D), jnp.int32),
        ],
    )(wdx, wrx, dxa1, dxa2, dxb, rxa1, rxa2, rxb)


NBUF = 4


def _run_spec(tab, idx3d, out, idxbuf, rows, gsem, wsem, wid, k, ch):
    base = wid * k
    pltpu.sync_copy(idx3d.at[wid], idxbuf)

    def wait(par, sem):
        pltpu.make_async_copy(out.at[0], rows.at[par], sem.at[par]).wait()

    for i in range(min(NBUF, k)):
        pltpu.async_copy(tab.at[idxbuf.at[i]], rows.at[i], gsem.at[i])

    def body(c, carry):
        par = lax.rem(c, NBUF)
        wait(par, gsem)
        pltpu.async_copy(rows.at[par], out.at[base + c], wsem.at[par])

        @pl.when(c + NBUF < k)
        def _():
            wait(par, wsem)
            pltpu.async_copy(tab.at[idxbuf.at[c + NBUF]], rows.at[par],
                             gsem.at[par])
        return carry

    lax.fori_loop(0, k, body, 0)
    for c in range(max(0, k - NBUF), k):
        wait(c % NBUF, wsem)


def _sc_gather_body(keys, chunks, ch, *refs):
    n = len(keys)
    tabs = refs[:n]
    idxs = refs[n:2 * n]
    outs = refs[2 * n:3 * n]
    scratch = list(refs[3 * n:])
    wid = lax.axis_index("s") * 2 + lax.axis_index("c")
    idxbufs = scratch[:n]
    per_k = {}
    for kk in sorted(set(keys)):
        per_k[kk] = tuple(scratch[n:n + 3])
        del scratch[n:n + 3]
    for i in range(n):
        rows, gsem, wsem = per_k[keys[i]]
        _run_spec(tabs[i], idxs[i], outs[i], idxbufs[i], rows, gsem, wsem,
                  wid, chunks[i], ch)


def _sc_gathers(tables, idx_lists, ch):
    keys = tuple((int(t.shape[1]), str(t.dtype)) for t in tables)
    chunks = tuple(int(ix.shape[1]) for ix in idx_lists)
    scratch = [pltpu.VMEM((k, ch), jnp.int32) for k in chunks]
    for w, dt in sorted(set(keys)):
        scratch.append(pltpu.VMEM((NBUF, ch, w), jnp.dtype(dt)))
        scratch.append(pltpu.SemaphoreType.DMA((NBUF,)))
        scratch.append(pltpu.SemaphoreType.DMA((NBUF,)))
    out_type = [jax.ShapeDtypeStruct((NW * k, ch, w), jnp.dtype(dt))
                for k, (w, dt) in zip(chunks, keys)]
    mesh = plsc.VectorSubcoreMesh(core_axis_name="c", subcore_axis_name="s")
    k = pl.kernel(
        functools.partial(_sc_gather_body, keys, chunks, ch),
        out_type=out_type,
        mesh=mesh,
        scratch_types=scratch,
    )
    return k(*tables, *idx_lists)


def _t2_body(nvalid, lb, g1_ref, pa_ref, u_ref, ehr_ref, out_ref, pk_ref):
    g1 = g1_ref[...]
    pa = pa_ref[...]
    ea = lax.bitcast_convert_type(pa << 16, F32)
    p2 = lax.bitcast_convert_type(pa & jnp.int32(-65536), F32)
    t = jnp.tanh(g1 + p2)
    pre = jnp.sum(t * u_ref[...], axis=2)
    m = jnp.max(pre, axis=1, keepdims=True)
    e = jnp.exp(pre - m)
    attn = e / jnp.sum(e, axis=1, keepdims=True)
    res = jnp.sum(attn[:, :, None] * ea, axis=1)
    rid = pl.program_id(0) * lb + lax.broadcasted_iota(jnp.int32, (lb, 1), 0)
    res = jnp.where(rid < nvalid, res, 0.0)
    out_ref[...] = res
    pk_ref[...] = _pack2(ehr_ref[...], res)


def _t2(g1, pa, u, ehr, maxa, nvalid, lb=256):
    nl = g1.shape[0] // maxa
    g1 = g1.reshape(nl, maxa, ADP)
    pa = pa.reshape(nl, maxa, D)
    ehr = jnp.pad(ehr, ((0, nl - ehr.shape[0]), (0, 0)))
    grid = nl // lb
    return pl.pallas_call(
        functools.partial(_t2_body, nvalid, lb),
        grid=(grid,),
        in_specs=[
            pl.BlockSpec((lb, maxa, ADP), lambda i: (i, 0, 0)),
            pl.BlockSpec((lb, maxa, D), lambda i: (i, 0, 0)),
            pl.BlockSpec((1, 1, ADP), lambda i: (0, 0, 0)),
            pl.BlockSpec((lb, D), lambda i: (i, 0)),
        ],
        out_specs=[
            pl.BlockSpec((lb, D), lambda i: (i, 0)),
            pl.BlockSpec((lb, D), lambda i: (i, 0)),
        ],
        out_shape=[
            jax.ShapeDtypeStruct((nl, D), F32),
            jax.ShapeDtypeStruct((nl, D), jnp.int32),
        ],
    )(g1, pa, u.reshape(1, 1, ADP), ehr)


def _t3_body(ndx, pdx_ref, prx_ref, ehr_ref, onto_ref):
    pdx = pdx_ref[...]
    prx = prx_ref[...]
    dxe = lax.bitcast_convert_type(pdx << 16, F32)
    rxe = lax.bitcast_convert_type(prx << 16, F32)
    sdx = jnp.sum(dxe, axis=1)
    srx = jnp.sum(rxe, axis=1)
    vdx = sdx * lax.rsqrt(jnp.maximum(jnp.sum(sdx * sdx, axis=1, keepdims=True), 1e-24))
    vrx = srx * lax.rsqrt(jnp.maximum(jnp.sum(srx * srx, axis=1, keepdims=True), 1e-24))
    ehr_ref[...] = vdx + vrx
    onto_ref[:, :ndx, :] = lax.bitcast_convert_type(pdx & jnp.int32(-65536), F32)
    onto_ref[:, ndx:, :] = lax.bitcast_convert_type(prx & jnp.int32(-65536), F32)


def _t3(gpdx, gprx, ndx, nrx, rows, rb=64):
    gpdx = gpdx.reshape(rows, ndx, D)
    gprx = gprx.reshape(rows, nrx, D)
    nt = ndx + nrx
    return pl.pallas_call(
        functools.partial(_t3_body, ndx),
        grid=(rows // rb,),
        in_specs=[
            pl.BlockSpec((rb, ndx, D), lambda i: (i, 0, 0)),
            pl.BlockSpec((rb, nrx, D), lambda i: (i, 0, 0)),
        ],
        out_specs=[
            pl.BlockSpec((rb, D), lambda i: (i, 0)),
            pl.BlockSpec((rb, nt, D), lambda i: (i, 0, 0)),
        ],
        out_shape=[
            jax.ShapeDtypeStruct((rows, D), F32),
            jax.ShapeDtypeStruct((rows, nt, D), F32),
        ],
    )(gpdx, gprx)


def _t4_body(x_ref, w_ref, b_ref, out_ref):
    y = jnp.dot(x_ref[...], w_ref[...], preferred_element_type=F32) + b_ref[...]
    m = jnp.max(y, axis=1, keepdims=True)
    e = jnp.exp(y - m)
    out_ref[...] = e / jnp.sum(e, axis=1, keepdims=True)


def _t4(x, w, b, rb=64):
    rows, nv = x.shape[0], w.shape[1]
    return pl.pallas_call(
        _t4_body,
        grid=(rows // rb,),
        in_specs=[
            pl.BlockSpec((rb, D), lambda i: (i, 0)),
            pl.BlockSpec((D, nv), lambda i: (0, 0)),
            pl.BlockSpec((1, nv), lambda i: (0, 0)),
        ],
        out_specs=pl.BlockSpec((rb, nv), lambda i: (i, 0)),
        out_shape=jax.ShapeDtypeStruct((rows, nv), F32),
    )(x, w, b.reshape(1, nv))


def _t5_body(oh_ref, tab_ref, out_ref):
    out_ref[0] = jnp.dot(oh_ref[0], tab_ref[...], preferred_element_type=F32)


def _t5(onehot, table):
    v, b, nv = onehot.shape
    return pl.pallas_call(
        _t5_body,
        grid=(v,),
        in_specs=[
            pl.BlockSpec((1, b, nv), lambda i: (i, 0, 0)),
            pl.BlockSpec((nv, D), lambda i: (0, 0)),
        ],
        out_specs=pl.BlockSpec((1, b, D), lambda i: (i, 0, 0)),
        out_shape=jax.ShapeDtypeStruct((v, b, D), F32),
    )(onehot, table)


def kernel(dxseqs, drugseqs, dx_onehot, drug_onehot, dxLeavesList,
           dxAncestorsList, drugLeavesList, drugAncestorsList,
           ctd_dx_leaves_list, ctd_dx_ancesster_list, ctd_dx_rel_list,
           ctd_dx_permute_list, ctd_rx_leaves_list, ctd_rx_ancesster_list,
           ctd_rx_rel_list, ctd_rx_permute_list, EHRdxEmb_W, EHRdrugEmb_W,
           dxOntoW, drugOntoW, dxAttnW, dxAttnb, dxAttnU, drugAttnW,
           drugAttnb, drugAttnU, cooccurW, cooccurB):
    B, V, NDX = dxseqs.shape
    NRX = drugseqs.shape[2]
    DXV, MAXA = dxLeavesList.shape
    RXV = drugLeavesList.shape[0]
    AD = dxAttnW.shape[1]

    def padw(m):
        return jnp.pad(m, ((0, 0), (0, ADP - AD)))

    dxa1, dxa2 = padw(dxAttnW[:D]), padw(dxAttnW[D:])
    rxa1, rxa2 = padw(drugAttnW[:D]), padw(drugAttnW[D:])
    dxb = jnp.pad(dxAttnb, (0, ADP - AD)).reshape(1, ADP)
    rxb = jnp.pad(drugAttnb, (0, ADP - AD)).reshape(1, ADP)
    dxu = jnp.pad(dxAttnU[:, 0], (0, ADP - AD))
    rxu = jnp.pad(drugAttnU[:, 0], (0, ADP - AD))

    p1dx, padx, p1rx, parx = _t1(dxOntoW, drugOntoW, dxa1, dxa2, dxb,
                                 rxa1, rxa2, rxb)

    def flatpad(ix, n, ch):
        f = ix.reshape(-1).astype(jnp.int32)
        return jnp.pad(f, (0, n - f.shape[0])).reshape(NW, -1, ch)

    CH = 100
    npair_dx = _pad_rows(DXV * MAXA, CH)
    npair_rx = _pad_rows(RXV * MAXA, CH)
    nseq_dx = _pad_rows(B * V * NDX, CH)
    nseq_rx = _pad_rows(B * V * NRX, CH)
    seqdx_ix = flatpad(dxseqs, nseq_dx, CH)
    seqrx_ix = flatpad(drugseqs, nseq_rx, CH)
    g1dx, gpadx, g1rx, gparx = _sc_gathers(
        [p1dx, padx, p1rx, parx],
        [flatpad(dxLeavesList, npair_dx, CH), flatpad(dxAncestorsList, npair_dx, CH),
         flatpad(drugLeavesList, npair_rx, CH), flatpad(drugAncestorsList, npair_rx, CH)],
        ch=CH,
    )

    dxall, pkdx = _t2(g1dx.reshape(-1, ADP), gpadx.reshape(-1, D), dxu,
                      EHRdxEmb_W, MAXA, DXV)
    rxall, pkrx = _t2(g1rx.reshape(-1, ADP), gparx.reshape(-1, D), rxu,
                      EHRdrugEmb_W, MAXA, RXV)

    gpk_dx, gpk_rx = _sc_gathers(
        [pkdx, pkrx],
        [seqdx_ix, seqrx_ix],
        ch=CH,
    )

    EHRVEmb, onto = _t3(gpk_dx.reshape(-1, D)[:B * V * NDX],
                        gpk_rx.reshape(-1, D)[:B * V * NRX],
                        NDX, NRX, B * V)

    cooccurU = _t4(EHRVEmb, cooccurW, cooccurB).reshape(B, V, -1)

    dxontoV = _t5(dx_onehot, dxall[:DXV])
    rxontoV = _t5(drug_onehot, rxall[:RXV])

    ontoEmb = onto.reshape(B, V, NDX + NRX, D)

    return (cooccurU,
            EHRVEmb.reshape(B, V, D),
            ontoEmb,
            jnp.transpose(dxontoV, (1, 0, 2)),
            jnp.transpose(rxontoV, (1, 0, 2)))
